# baseline (speedup 1.0000x reference)
import functools

import jax
import jax.numpy as jnp
from jax import lax
from jax.experimental import pallas as pl
from jax.experimental.pallas import tpu as pltpu
from jax.experimental.pallas import tpu_sc as plsc

_N = 4096
_R = 256
_G = _N // _R

_NC = 2
_NS = 16
_NW = _NC * _NS
_PW = _N // _NW


def _sc_diag_body(preds_hbm, d_out, blk_v, dval_v):
    wid = lax.axis_index("s") * _NC + lax.axis_index("c")
    base = wid * _PW
    lane = lax.iota(jnp.int32, 16)
    for j in range(_PW // 16):
        b0 = base + j * 16
        pltpu.sync_copy(preds_hbm.at[pl.ds(b0, 16), pl.ds(base, _PW)], blk_v)
        acc = jnp.zeros((16,), jnp.float32)
        for l in range(16):
            acc = jnp.where(lane == l, blk_v[l, pl.ds(j * 16, 16)], acc)
        dval_v[pl.ds(j * 16, 16)] = acc
    pltpu.sync_copy(dval_v, d_out.at[pl.ds(base, _PW)])


_sc_diag_cache = []


def _sc_diag(preds):
    if not _sc_diag_cache:
        _sc_diag_cache.append(functools.partial(
            pl.kernel,
            mesh=plsc.VectorSubcoreMesh(core_axis_name="c", subcore_axis_name="s"),
            out_type=jax.ShapeDtypeStruct((_N,), jnp.float32),
            scratch_types=[
                pltpu.VMEM((16, _PW), jnp.float32),
                pltpu.VMEM((_PW,), jnp.float32),
            ],
        )(_sc_diag_body))
    return _sc_diag_cache[0](preds)


_K = 32
_W = _N // _K
_LOG2E = 1.4426950408889634


def _tc_body(x_ref, tgt_ref, d_ref, out_ref):
    i = pl.program_id(0)
    neg = jnp.float32(-jnp.inf)
    tb = tgt_ref[...]
    d = d_ref[...]
    kb = lax.shift_right_logical(tb, 7)
    cb = jnp.bitwise_and(tb, _W - 1)

    A = x_ref[:, 0:_W]
    tsel = A
    for k in range(1, _K):
        xk = x_ref[:, k * _W:(k + 1) * _W]
        A = jnp.maximum(A, xk)
        tsel = jnp.where(kb == k, xk, tsel)
    m0 = jnp.max(A, axis=1, keepdims=True)

    b = m0 * jnp.float32(-_LOG2E)
    M2 = neg
    epart = jnp.zeros((_R, _W), jnp.float32)
    for k in range(_K):
        xk = x_ref[:, k * _W:(k + 1) * _W]
        M2 = jnp.maximum(M2, jnp.where(xk < A, xk, neg))
        epart = epart + jnp.exp2(xk * jnp.float32(_LOG2E) + b)
    s = jnp.sum(epart, axis=1, keepdims=True)
    lse = m0 + jnp.log(s)

    lane = lax.broadcasted_iota(jnp.int32, (_R, _W), 1)
    t = jnp.sum(jnp.where(lane == cb, tsel, 0.0), axis=1, keepdims=True)
    m1 = jnp.maximum(
        jnp.max(jnp.where(A < m0, A, neg), axis=1, keepdims=True),
        jnp.max(M2, axis=1, keepdims=True))
    m2 = jnp.maximum(
        jnp.max(jnp.where(A < m1, A, neg), axis=1, keepdims=True),
        jnp.max(jnp.where(M2 < m1, M2, neg), axis=1, keepdims=True))

    rowid = i * _R + lax.broadcasted_iota(jnp.int32, (_R, 1), 0)
    is0 = d == m0
    is1 = d == m1
    va = jnp.where(is0, m1, m0)
    vb = jnp.where(is0 | is1, m2, m1)
    ind = ((tb != rowid) & (t != va) & (t != vb)).astype(jnp.float32)
    loss = lse * (1.0 + ind) - (0.7 * d + 0.2 * va + 0.1 * vb + ind * t)
    part = jnp.sum(loss, axis=0, keepdims=True) * jnp.float32(1.0 / _N)
    prev = jnp.where(i == 0, jnp.zeros_like(part), out_ref[...])
    out_ref[...] = prev + part


def kernel(preds, targets):
    tgt = targets.astype(jnp.int32)
    d = _sc_diag(preds)
    out = pl.pallas_call(
        _tc_body,
        grid=(_G,),
        in_specs=[
            pl.BlockSpec((_R, _N), lambda i: (i, 0)),
            pl.BlockSpec((_R, 1), lambda i: (i, 0)),
            pl.BlockSpec((_R, 1), lambda i: (i, 0)),
        ],
        out_specs=pl.BlockSpec((1, 1), lambda i: (0, 0)),
        out_shape=jax.ShapeDtypeStruct((1, 1), jnp.float32),
    )(preds, tgt.reshape(_N, 1), d.reshape(_N, 1))
    return out[0, 0]
